# baseline (speedup 1.0000x reference)
import functools

import jax
import jax.numpy as jnp
from jax import lax
from jax.experimental import pallas as pl
from jax.experimental.pallas import tpu as pltpu
from jax.experimental.pallas import tpu_sc as plsc

NUM_EMB = 1000000
D = 64
B = 16384
TOTW = 2 * B
L = 16
NC, NS = 2, 16
NW = NC * NS
BPW = B // NW
NCOL = (NUM_EMB + 127) // 128
NBLK = 248
SCAP = 4160
SCNT = 4096
BCAP = 576
BCNT = 512
DUMMY = TOTW
NROWS = TOTW + L


def _rsqrt(x):
    i = plsc.bitcast(x, jnp.int32)
    i = jnp.int32(0x5F3759DF) - (i >> 1)
    y = plsc.bitcast(i, jnp.float32)
    for _ in range(3):
        y = y * (1.5 - 0.5 * x * y * y)
    return y


_PARAMS = dict(
    needs_layout_passes=False,
    use_tc_tiling_on_sc=True,
    disable_bounds_checks=True,
)


@functools.cache
def _build_k1():
    @functools.partial(
        pl.kernel,
        out_type=jax.ShapeDtypeStruct((NROWS, 128), jnp.float32),
        mesh=plsc.VectorSubcoreMesh(core_axis_name="c", subcore_axis_name="s"),
        compiler_params=pltpu.CompilerParams(**_PARAMS),
        scratch_types=[
            pltpu.VMEM((B,), jnp.int32),
            pltpu.VMEM((B,), jnp.int32),
            pltpu.VMEM((D, 128), jnp.float32),
            pltpu.VMEM((D, 128), jnp.float32),
            pltpu.VMEM((SCAP,), jnp.int32),
            pltpu.VMEM((SCAP,), jnp.int32),
            pltpu.VMEM((BCAP,), jnp.int32),
            pltpu.VMEM((BCAP,), jnp.int32),
            pltpu.VMEM((L, 128), jnp.float32),
            pltpu.VMEM((L, 128), jnp.float32),
            pltpu.SemaphoreType.DMA,
            pltpu.SemaphoreType.DMA,
            pltpu.SemaphoreType.DMA,
            pltpu.SemaphoreType.DMA,
        ],
    )
    def k1(w1_hbm, w2_hbm, tT_hbm, rows_hbm,
           wb1, wb2, blk0, blk1, swl, spl, bwl, bpl, st0, st1,
           sb0, sb1, ss0, ss1):
        blks = (blk0, blk1)
        sbs = (sb0, sb1)
        sts = (st0, st1)
        sss = (ss0, ss1)
        iota = lax.iota(jnp.int32, L)
        wid = lax.axis_index("s") * NC + lax.axis_index("c")
        cs = (wid * NCOL) // NW
        ce = ((wid + 1) * NCOL) // NW

        def blk_col(b):
            return jnp.minimum(cs + b, ce - 1)

        def blk_off(b):
            return pl.multiple_of(blk_col(b) * 128, 128)

        for sl in range(2):
            pltpu.async_copy(
                tT_hbm.at[:, pl.ds(blk_off(sl), 128)], blks[sl], sbs[sl])

        pltpu.sync_copy(w1_hbm, wb1)
        pltpu.sync_copy(w2_hbm, wb2)
        lo = cs * 128
        hi = ce * 128

        def scan(wb, pos0):
            def body(i, cnt):
                wv = wb[pl.ds(i * L, L)]
                m = (wv >= lo) & (wv < hi)
                plsc.store_compressed(swl.at[pl.ds(cnt, L)], wv, mask=m)
                plsc.store_compressed(
                    spl.at[pl.ds(cnt, L)], iota + (pos0 + i * L), mask=m)
                c = plsc.all_reduce_population_count(m)[0]
                return jnp.minimum(cnt + c, SCNT)
            return body

        cnt = lax.fori_loop(0, B // L, scan(wb1, 0), 0)
        cnt = lax.fori_loop(0, B // L, scan(wb2, B), cnt)
        swl[pl.ds(cnt, L)] = jnp.zeros((L,), jnp.int32) + lo
        spl[pl.ds(cnt, L)] = jnp.full((L,), DUMMY, jnp.int32)
        nch = (cnt + L - 1) // L

        dummy_idx = jnp.full((L,), DUMMY, jnp.int32)
        for sl in range(2):
            pltpu.async_copy(sts[sl], rows_hbm.at[dummy_idx], sss[sl])

        def process_block(b, sl):
            col = blk_col(b)
            blk = blks[sl]

            def rescan(i, bcnt):
                wv = swl[pl.ds(i * L, L)]
                pv = spl[pl.ds(i * L, L)]
                m = (wv >> 7) == col
                plsc.store_compressed(bwl.at[pl.ds(bcnt, L)], wv, mask=m)
                plsc.store_compressed(bpl.at[pl.ds(bcnt, L)], pv, mask=m)
                c = plsc.all_reduce_population_count(m)[0]
                return jnp.minimum(bcnt + c, BCNT)

            bcnt = lax.fori_loop(0, nch, rescan, 0)
            wpad = jnp.zeros((L,), jnp.int32) + col * 128
            ppad = jnp.full((L,), DUMMY, jnp.int32)
            bwl[pl.ds(bcnt, L)] = wpad
            bpl[pl.ds(bcnt, L)] = ppad
            bwl[pl.ds(bcnt + L, L)] = wpad
            bpl[pl.ds(bcnt + L, L)] = ppad
            npair = (bcnt + 2 * L - 1) // (2 * L)

            def extract(h, carry):
                for ssl in range(2):
                    ci = h * 2 + ssl
                    wv = bwl[pl.ds(ci * L, L)]
                    pv = bpl[pl.ds(ci * L, L)]
                    pltpu.make_async_copy(
                        sts[ssl], rows_hbm.at[pl.ds(TOTW, L)],
                        sss[ssl]).wait()
                    for j in range(L):
                        lvec = jnp.zeros((L,), jnp.int32) + (wv[j] - col * 128)
                        for dc in range(D // L):
                            g = plsc.load_gather(
                                blk, [iota + dc * L, lvec])
                            sts[ssl][j, pl.ds(dc * L, L)] = g
                    pltpu.async_copy(sts[ssl], rows_hbm.at[pv], sss[ssl])
                return carry

            lax.fori_loop(0, npair, extract, 0)

        def ring(g, carry):
            for sl in range(2):
                b = g * 2 + sl
                pltpu.make_async_copy(
                    tT_hbm.at[:, pl.ds(0, 128)], blks[sl], sbs[sl]).wait()
                process_block(b, sl)
                pltpu.async_copy(
                    tT_hbm.at[:, pl.ds(blk_off(b + 2), 128)],
                    blks[sl], sbs[sl])
            return carry

        lax.fori_loop(0, NBLK // 2, ring, 0)

        for sl in range(2):
            pltpu.make_async_copy(
                tT_hbm.at[:, pl.ds(0, 128)], blks[sl], sbs[sl]).wait()
            pltpu.make_async_copy(
                sts[sl], rows_hbm.at[pl.ds(TOTW, L)], sss[sl]).wait()

    return k1


@functools.cache
def _build_k2():
    @functools.partial(
        pl.kernel,
        out_type=jax.ShapeDtypeStruct((B,), jnp.float32),
        mesh=plsc.VectorSubcoreMesh(core_axis_name="c", subcore_axis_name="s"),
        compiler_params=pltpu.CompilerParams(**_PARAMS),
        scratch_types=[
            pltpu.VMEM((BPW // 2, 128), jnp.float32),
            pltpu.VMEM((BPW // 2, 128), jnp.float32),
            pltpu.VMEM((L * 24,), jnp.float32),
            pltpu.VMEM((L * 24,), jnp.float32),
            pltpu.VMEM((L * 24,), jnp.float32),
            pltpu.VMEM((BPW,), jnp.float32),
        ],
    )
    def k2(rows_hbm, out_hbm, r1v, r2v, dot_v, n1_v, n2_v, out_v):
        H = BPW // 2
        iota = lax.iota(jnp.int32, L)
        wid = lax.axis_index("s") * NC + lax.axis_index("c")
        base = wid * BPW

        for hb in range(2):
            pltpu.sync_copy(rows_hbm.at[pl.ds(base + hb * H, H), :], r1v)
            pltpu.sync_copy(rows_hbm.at[pl.ds(B + base + hb * H, H), :], r2v)

            def group_body(g, carry):
                for i in range(L):
                    p = g * L + i
                    a0 = r1v[p, pl.ds(0 * L, L)]
                    a1 = r1v[p, pl.ds(1 * L, L)]
                    a2 = r1v[p, pl.ds(2 * L, L)]
                    a3 = r1v[p, pl.ds(3 * L, L)]
                    b0 = r2v[p, pl.ds(0 * L, L)]
                    b1 = r2v[p, pl.ds(1 * L, L)]
                    b2 = r2v[p, pl.ds(2 * L, L)]
                    b3 = r2v[p, pl.ds(3 * L, L)]
                    dot_v[pl.ds(i * 24, L)] = a0 * b0 + a1 * b1 + a2 * b2 + a3 * b3
                    n1_v[pl.ds(i * 24, L)] = a0 * a0 + a1 * a1 + a2 * a2 + a3 * a3
                    n2_v[pl.ds(i * 24, L)] = b0 * b0 + b1 * b1 + b2 * b2 + b3 * b3
                rowbase = iota * 24
                acc_d = plsc.load_gather(dot_v, [rowbase])
                acc_1 = plsc.load_gather(n1_v, [rowbase])
                acc_2 = plsc.load_gather(n2_v, [rowbase])
                for j in range(1, L):
                    cj = rowbase + j
                    acc_d = acc_d + plsc.load_gather(dot_v, [cj])
                    acc_1 = acc_1 + plsc.load_gather(n1_v, [cj])
                    acc_2 = acc_2 + plsc.load_gather(n2_v, [cj])
                s1 = jnp.maximum(acc_1, 1e-16)
                s2 = jnp.maximum(acc_2, 1e-16)
                out_v[pl.ds(hb * H + g * L, L)] = acc_d * _rsqrt(s1) * _rsqrt(s2)
                return carry

            lax.fori_loop(0, H // L, group_body, 0)

        pltpu.sync_copy(out_v, out_hbm.at[pl.ds(base, BPW)])

    return k2


def kernel(words1, words2, table):
    w1 = words1.astype(jnp.int32)
    w2 = words2.astype(jnp.int32)
    rows = _build_k1()(w1, w2, table.T)
    return _build_k2()(rows)


# speedup vs baseline: 14.0303x; 14.0303x over previous
import functools

import jax
import jax.numpy as jnp
from jax import lax
from jax.experimental import pallas as pl
from jax.experimental.pallas import tpu as pltpu
from jax.experimental.pallas import tpu_sc as plsc

NUM_EMB = 1000000
D = 64
B = 16384
L = 16
NC, NS = 2, 16
NW = NC * NS
BPW = B // NW
GCH = 128
NG = BPW // GCH


def _rsqrt(x):
    i = plsc.bitcast(x, jnp.int32)
    i = jnp.int32(0x5F3759DF) - (i >> 1)
    y = plsc.bitcast(i, jnp.float32)
    for _ in range(3):
        y = y * (1.5 - 0.5 * x * y * y)
    return y


@functools.cache
def _build():
    @functools.partial(
        pl.kernel,
        out_type=jax.ShapeDtypeStruct((B,), jnp.float32),
        mesh=plsc.VectorSubcoreMesh(core_axis_name="c", subcore_axis_name="s"),
        compiler_params=pltpu.CompilerParams(
            needs_layout_passes=False, use_tc_tiling_on_sc=False),
        scratch_types=[
            pltpu.VMEM((NG, GCH), jnp.int32),
            pltpu.VMEM((NG, GCH), jnp.int32),
            pltpu.VMEM((BPW, D), jnp.float32),
            pltpu.VMEM((BPW, D), jnp.float32),
            pltpu.VMEM((L, L + 1), jnp.float32),
            pltpu.VMEM((L, L + 1), jnp.float32),
            pltpu.VMEM((L, L + 1), jnp.float32),
            pltpu.VMEM((BPW,), jnp.float32),
            pltpu.SemaphoreType.DMA,
        ],
    )
    def _cos_sim_sc(w1_hbm, w2_hbm, table_hbm, out_hbm,
                    idx1_v, idx2_v, rows1_v, rows2_v,
                    dot_v, n1_v, n2_v, out_v, sem):
        wid = lax.axis_index("s") * NC + lax.axis_index("c")
        base = wid * BPW

        pltpu.sync_copy(w1_hbm.at[wid], idx1_v)
        pltpu.sync_copy(w2_hbm.at[wid], idx2_v)

        cps = []
        for j in range(NG):
            cps.append(pltpu.async_copy(
                table_hbm.at[idx1_v.at[j]],
                rows1_v.at[pl.ds(j * GCH, GCH)], sem))
            cps.append(pltpu.async_copy(
                table_hbm.at[idx2_v.at[j]],
                rows2_v.at[pl.ds(j * GCH, GCH)], sem))
        for cp in cps:
            cp.wait()

        iota = lax.iota(jnp.int32, L)

        def group_body(g, carry):
            for i in range(L):
                p = g * L + i
                a0 = rows1_v[p, pl.ds(0 * L, L)]
                a1 = rows1_v[p, pl.ds(1 * L, L)]
                a2 = rows1_v[p, pl.ds(2 * L, L)]
                a3 = rows1_v[p, pl.ds(3 * L, L)]
                b0 = rows2_v[p, pl.ds(0 * L, L)]
                b1 = rows2_v[p, pl.ds(1 * L, L)]
                b2 = rows2_v[p, pl.ds(2 * L, L)]
                b3 = rows2_v[p, pl.ds(3 * L, L)]
                dot_v[i, pl.ds(0, L)] = a0 * b0 + a1 * b1 + a2 * b2 + a3 * b3
                n1_v[i, pl.ds(0, L)] = a0 * a0 + a1 * a1 + a2 * a2 + a3 * a3
                n2_v[i, pl.ds(0, L)] = b0 * b0 + b1 * b1 + b2 * b2 + b3 * b3
            acc_d = plsc.load_gather(dot_v, [iota, jnp.full((L,), 0, jnp.int32)])
            acc_1 = plsc.load_gather(n1_v, [iota, jnp.full((L,), 0, jnp.int32)])
            acc_2 = plsc.load_gather(n2_v, [iota, jnp.full((L,), 0, jnp.int32)])
            for j in range(1, L):
                cj = jnp.full((L,), j, jnp.int32)
                acc_d = acc_d + plsc.load_gather(dot_v, [iota, cj])
                acc_1 = acc_1 + plsc.load_gather(n1_v, [iota, cj])
                acc_2 = acc_2 + plsc.load_gather(n2_v, [iota, cj])
            s1 = jnp.maximum(acc_1, 1e-16)
            s2 = jnp.maximum(acc_2, 1e-16)
            out_v[pl.ds(g * L, L)] = acc_d * _rsqrt(s1) * _rsqrt(s2)
            return carry

        lax.fori_loop(0, BPW // L, group_body, 0)

        pltpu.sync_copy(out_v, out_hbm.at[pl.ds(base, BPW)])

    return _cos_sim_sc


def kernel(words1, words2, table):
    w1 = words1.astype(jnp.int32).reshape(NW, NG, GCH)
    w2 = words2.astype(jnp.int32).reshape(NW, NG, GCH)
    return _build()(w1, w2, table)
